# baseline (speedup 1.0000x reference)
import functools

import jax
import jax.numpy as jnp
from jax import lax
from jax.experimental import pallas as pl
from jax.experimental.pallas import tpu as pltpu
from jax.experimental.pallas import tpu_sc as plsc

_N = 10000
_E = 320000
_NC, _NS = 2, 16
_NW = _NC * _NS
_EPW = _E // _NW
_NP = 10240
_STRIPE = _NP // _NS


def _make_mp(D: int, chunk: int):
    nchunks = _EPW // chunk
    mesh = plsc.VectorSubcoreMesh(
        core_axis_name="c", subcore_axis_name="s",
        num_cores=_NC, num_subcores=_NS)

    @functools.partial(
        pl.kernel,
        mesh=mesh,
        out_type=jax.ShapeDtypeStruct((_NC * _NP, D), jnp.float32),
        scratch_types=[
            pltpu.VMEM((nchunks, chunk), jnp.int32),
            pltpu.VMEM((nchunks, chunk), jnp.int32),
            pltpu.VMEM((chunk, D), jnp.float32),
            pltpu.VMEM((chunk, D), jnp.float32),
            pltpu.VMEM_SHARED((_NP, D), jnp.float32),
            pltpu.SemaphoreType.DMA,
            pltpu.SemaphoreType.DMA,
            pltpu.SemaphoreType.DMA,
            pltpu.SemaphoreType.DMA,
        ],
        compiler_params=pltpu.CompilerParams(use_tc_tiling_on_sc=False),
    )
    def mp(h_hbm, src_hbm, dst_hbm, zero_hbm, out_hbm,
           src_v, dst_v, rows0, rows1, acc_sh, g0, g1, s0, s1):
        c = lax.axis_index("c")
        s = lax.axis_index("s")
        wid = c * _NS + s
        bufs = (rows0, rows1)
        gsem = (g0, g1)
        ssem = (s0, s1)
        pltpu.sync_copy(src_hbm.at[wid], src_v)
        pltpu.sync_copy(dst_hbm.at[wid], dst_v)
        pltpu.sync_copy(zero_hbm.at[pl.ds(s * _STRIPE, _STRIPE)],
                        acc_sh.at[pl.ds(s * _STRIPE, _STRIPE)])
        plsc.subcore_barrier()

        gather = [None, None]
        scatter = [None, None]
        gather[0] = pltpu.async_copy(h_hbm.at[src_v.at[0]], bufs[0], gsem[0])
        for j in range(nchunks):
            b = j & 1
            gather[b].wait()
            scatter[b] = pltpu.async_copy(
                bufs[b], acc_sh.at[dst_v.at[j]], ssem[b], add=True)
            if j + 1 < nchunks:
                if scatter[1 - b] is not None:
                    scatter[1 - b].wait()
                gather[1 - b] = pltpu.async_copy(
                    h_hbm.at[src_v.at[j + 1]], bufs[1 - b], gsem[1 - b])
        for d in scatter:
            if d is not None:
                d.wait()
        plsc.subcore_barrier()
        pltpu.sync_copy(acc_sh.at[pl.ds(s * _STRIPE, _STRIPE)],
                        out_hbm.at[pl.ds(c * _NP + s * _STRIPE, _STRIPE)])

    return mp


_CHUNK = 1250
_NCHUNKS = _EPW // _CHUNK
_mp32 = _make_mp(32, _CHUNK)
_mp16 = _make_mp(16, _CHUNK)


def _mm1_body(x_ref, w_ref, o_ref):
    o_ref[...] = jnp.dot(x_ref[...], w_ref[...],
                         preferred_element_type=jnp.float32)


def _mm2_body(p_ref, w_ref, o_ref):
    h = jax.nn.relu(p_ref[:_N, :] + p_ref[_NP:_NP + _N, :])
    o_ref[...] = jnp.dot(h, w_ref[...], preferred_element_type=jnp.float32)


def _softmax_body(p_ref, o_ref):
    z = p_ref[:_N, :] + p_ref[_NP:_NP + _N, :]
    z = z - jnp.max(z, axis=-1, keepdims=True)
    e = jnp.exp(z)
    o_ref[...] = e / jnp.sum(e, axis=-1, keepdims=True)


def kernel(x, edge_index, W1, W2):
    src = edge_index[0].astype(jnp.int32).reshape(_NW, _NCHUNKS, _CHUNK)
    dst = edge_index[1].astype(jnp.int32).reshape(_NW, _NCHUNKS, _CHUNK)
    z32 = jnp.zeros((_NP, 32), jnp.float32)
    z16 = jnp.zeros((_NP, 16), jnp.float32)

    h1pre = pl.pallas_call(
        _mm1_body,
        out_shape=jax.ShapeDtypeStruct((_N, 32), jnp.float32),
    )(x, W1)

    m1 = _mp32(h1pre, src, dst, z32)

    h2pre = pl.pallas_call(
        _mm2_body,
        out_shape=jax.ShapeDtypeStruct((_N, 16), jnp.float32),
    )(m1, W2)

    m2 = _mp16(h2pre, src, dst, z16)

    out = pl.pallas_call(
        _softmax_body,
        out_shape=jax.ShapeDtypeStruct((_N, 16), jnp.float32),
    )(m2)
    return out


# speedup vs baseline: 1.0309x; 1.0309x over previous
import functools

import jax
import jax.numpy as jnp
from jax import lax
from jax.experimental import pallas as pl
from jax.experimental.pallas import tpu as pltpu
from jax.experimental.pallas import tpu_sc as plsc

_N = 10000
_NBUF = 3
_E = 320000
_NC, _NS = 2, 16
_NW = _NC * _NS
_EPW = _E // _NW
_NP = 10240
_STRIPE = _NP // _NS


def _make_mp(D: int, chunk: int):
    nchunks = _EPW // chunk
    mesh = plsc.VectorSubcoreMesh(
        core_axis_name="c", subcore_axis_name="s",
        num_cores=_NC, num_subcores=_NS)

    @functools.partial(
        pl.kernel,
        mesh=mesh,
        out_type=jax.ShapeDtypeStruct((_NC * _NP, D), jnp.float32),
        scratch_types=[
            pltpu.VMEM((nchunks, chunk), jnp.int32),
            pltpu.VMEM((nchunks, chunk), jnp.int32),
            [pltpu.VMEM((chunk, D), jnp.float32) for _ in range(_NBUF)],
            [pltpu.SemaphoreType.DMA for _ in range(_NBUF)],
            [pltpu.SemaphoreType.DMA for _ in range(_NBUF)],
            pltpu.VMEM_SHARED((_NP, D), jnp.float32),
        ],
        compiler_params=pltpu.CompilerParams(use_tc_tiling_on_sc=False),
    )
    def mp(h_hbm, src_hbm, dst_hbm, zero_hbm, out_hbm,
           src_v, dst_v, bufs, gsem, ssem, acc_sh):
        c = lax.axis_index("c")
        s = lax.axis_index("s")
        wid = c * _NS + s
        pltpu.sync_copy(src_hbm.at[wid], src_v)
        pltpu.sync_copy(dst_hbm.at[wid], dst_v)
        pltpu.sync_copy(zero_hbm.at[pl.ds(s * _STRIPE, _STRIPE)],
                        acc_sh.at[pl.ds(s * _STRIPE, _STRIPE)])
        plsc.subcore_barrier()

        gather = [None] * _NBUF
        scatter = [None] * _NBUF
        for j in range(min(_NBUF, nchunks)):
            gather[j] = pltpu.async_copy(
                h_hbm.at[src_v.at[j]], bufs[j], gsem[j])
        for j in range(nchunks):
            b = j % _NBUF
            gather[b].wait()
            scatter[b] = pltpu.async_copy(
                bufs[b], acc_sh.at[dst_v.at[j]], ssem[b], add=True)
            nj = j + _NBUF
            if nj < nchunks:
                scatter[b].wait()
                scatter[b] = None
                gather[b] = pltpu.async_copy(
                    h_hbm.at[src_v.at[nj]], bufs[b], gsem[b])
        for d in scatter:
            if d is not None:
                d.wait()
        plsc.subcore_barrier()
        pltpu.sync_copy(acc_sh.at[pl.ds(s * _STRIPE, _STRIPE)],
                        out_hbm.at[pl.ds(c * _NP + s * _STRIPE, _STRIPE)])

    return mp


_CHUNK = 625
_NCHUNKS = _EPW // _CHUNK
_mp32 = _make_mp(32, _CHUNK)
_mp16 = _make_mp(16, _CHUNK)


def _mm1_body(x_ref, w_ref, o_ref):
    o_ref[...] = jnp.dot(x_ref[...], w_ref[...],
                         preferred_element_type=jnp.float32)


def _mm2_body(p_ref, w_ref, o_ref):
    h = jax.nn.relu(p_ref[:_N, :] + p_ref[_NP:_NP + _N, :])
    o_ref[...] = jnp.dot(h, w_ref[...], preferred_element_type=jnp.float32)


def _softmax_body(p_ref, o_ref):
    z = p_ref[:_N, :] + p_ref[_NP:_NP + _N, :]
    z = z - jnp.max(z, axis=-1, keepdims=True)
    e = jnp.exp(z)
    o_ref[...] = e / jnp.sum(e, axis=-1, keepdims=True)


def kernel(x, edge_index, W1, W2):
    src = edge_index[0].astype(jnp.int32).reshape(_NW, _NCHUNKS, _CHUNK)
    dst = edge_index[1].astype(jnp.int32).reshape(_NW, _NCHUNKS, _CHUNK)
    z32 = jnp.zeros((_NP, 32), jnp.float32)
    z16 = jnp.zeros((_NP, 16), jnp.float32)

    h1pre = pl.pallas_call(
        _mm1_body,
        out_shape=jax.ShapeDtypeStruct((_N, 32), jnp.float32),
    )(x, W1)

    m1 = _mp32(h1pre, src, dst, z32)

    h2pre = pl.pallas_call(
        _mm2_body,
        out_shape=jax.ShapeDtypeStruct((_N, 16), jnp.float32),
    )(m1, W2)

    m2 = _mp16(h2pre, src, dst, z16)

    out = pl.pallas_call(
        _softmax_body,
        out_shape=jax.ShapeDtypeStruct((_N, 16), jnp.float32),
    )(m2)
    return out
